# Initial kernel scaffold; baseline (speedup 1.0000x reference)
import jax
import jax.numpy as jnp
from jax.experimental import pallas as pl


def kernel(scores, segments, labels):
    raise NotImplementedError("write your pallas kernel here")


# speedup vs baseline: 5.1218x; 5.1218x over previous
import functools

import jax
import jax.numpy as jnp
from jax import lax
from jax.experimental import pallas as pl
from jax.experimental.pallas import tpu as pltpu
from jax.experimental.pallas import tpu_sc as plsc

N = 20000
NV = N // 16
NLBL = 200
NCORES = 2
NSUB = 16
NW = NCORES * NSUB
LPW = 7
ML = NW * LPW
MLV = ML // 16
CAP = 224
K = 200
BIG = 1 << 30

def _wid():
    return lax.axis_index("s") * NCORES + lax.axis_index("c")


def _k1_body(smin_hbm, smax_hbm, lmin_hbm, lmax_hbm, lists_hbm, counts_hbm,
             smin_v, smax_v, lmin_v, lmax_v, list_v, cnt_v):
    w = _wid()
    base_l = w * LPW
    pltpu.sync_copy(smin_hbm, smin_v)
    pltpu.sync_copy(smax_hbm, smax_v)
    pltpu.sync_copy(lmin_hbm.at[pl.ds(base_l * 16, LPW * 16)], lmin_v)
    pltpu.sync_copy(lmax_hbm.at[pl.ds(base_l * 16, LPW * 16)], lmax_v)
    lanes = lax.iota(jnp.int32, 16)
    for j in range(LPW):
        bmin = lmin_v[pl.ds(j * 16, 16)]
        bmax = lmax_v[pl.ds(j * 16, 16)]
        blen = bmax - bmin

        def body(i, off):
            b = i * 16
            sm = smin_v[pl.ds(b, 16)]
            sx = smax_v[pl.ds(b, 16)]
            inter = jnp.maximum(
                jnp.minimum(sx, bmax) - jnp.maximum(sm, bmin), 0.0)
            union = (sx - sm) + blen - inter
            m = (inter + inter) > union
            moff = jnp.minimum(off, K)
            mstore = jnp.logical_and(m, off < K)
            mi = mstore.astype(jnp.int32)
            dest = moff + plsc.cumsum(mi) - mi
            plsc.store_scatter(list_v, [dest], lanes + b, mask=mstore)
            return off + jnp.sum(m.astype(jnp.int32))

        off = lax.fori_loop(0, NV, body, jnp.int32(0))
        pltpu.sync_copy(list_v, lists_hbm.at[pl.ds((base_l + j) * CAP, CAP)])
        cnt_v[pl.ds(j * 16, 16)] = jnp.full((16,), off, jnp.int32)
    pltpu.sync_copy(cnt_v, counts_hbm.at[pl.ds(w * LPW * 16, LPW * 16)])


def _k2_body(lists_hbm, counts_hbm, scores_hbm, chosen_hbm, svals_hbm,
             lists_v, counts_v, scores_v, taken_v, chosen_v, svals_v):
    w = _wid()

    @pl.when(w == 0)
    def _():
        pltpu.sync_copy(lists_hbm, lists_v)
        pltpu.sync_copy(counts_hbm, counts_v)
        pltpu.sync_copy(scores_hbm, scores_v)
        lanes = lax.iota(jnp.int32, 16)
        zeros = jnp.zeros((16,), jnp.int32)
        ones = jnp.ones((16,), jnp.int32)

        def zbody(i, carry):
            taken_v[pl.ds(i * 16, 16)] = zeros
            return carry

        lax.fori_loop(0, NV, zbody, 0)

        def lbody(c, carry):
            cnt = jnp.max(counts_v[pl.ds(c * 16, 16)])
            cntcap = jnp.minimum(cnt, K)

            def body(i, ch):
                candv = lists_v[pl.ds(c * CAP + i * 16, 16)]
                validm = (lanes + i * 16) < cntcap
                csafe = jnp.where(validm, candv, 0)
                tk = plsc.load_gather(taken_v, [csafe])
                avail = jnp.logical_and(validm, tk == 0)
                chn = jnp.min(jnp.where(avail, candv, BIG))
                chn = jnp.where(chn < BIG, chn, -1)
                return jnp.where(ch >= 0, ch, chn)

            ch = lax.fori_loop(0, (K + 15) // 16, body, jnp.int32(-1))
            has = ch >= 0
            chv = jnp.full((16,), ch, jnp.int32)
            csafe = jnp.maximum(chv, 0)
            plsc.store_scatter(taken_v, [csafe], ones,
                               mask=jnp.logical_and(lanes == 0, has))
            sv = plsc.load_gather(scores_v, [csafe])
            chosen_v[pl.ds(c * 16, 16)] = chv
            svals_v[pl.ds(c * 16, 16)] = jnp.where(has, sv, 0.0)
            return carry

        lax.fori_loop(0, ML, lbody, 0)
        pltpu.sync_copy(chosen_v, chosen_hbm)
        pltpu.sync_copy(svals_v, svals_hbm)


def _k3_body(scores_hbm, chosen_hbm, svals_hbm, ranks_hbm,
             scores_v, ch_v, sv_v, rk_v):
    w = _wid()
    pltpu.sync_copy(scores_hbm, scores_v)
    pltpu.sync_copy(chosen_hbm.at[pl.ds(w * LPW * 16, LPW * 16)], ch_v)
    pltpu.sync_copy(svals_hbm.at[pl.ds(w * LPW * 16, LPW * 16)], sv_v)
    lanes = lax.iota(jnp.int32, 16)
    for j in range(LPW):
        jv = ch_v[pl.ds(j * 16, 16)]
        sv = sv_v[pl.ds(j * 16, 16)]
        has = jnp.max(jv) >= 0

        def body(i, acc):
            b = i * 16
            sc = scores_v[pl.ds(b, 16)]
            idxv = lanes + b
            m = jnp.logical_or(
                sc > sv, jnp.logical_and(sc == sv, idxv < jv))
            return acc + m.astype(jnp.int32)

        acc = lax.fori_loop(0, NV, body, jnp.zeros((16,), jnp.int32))
        rank = jnp.where(has, jnp.sum(acc), BIG)
        rk_v[pl.ds(j * 16, 16)] = jnp.full((16,), rank, jnp.int32)
    pltpu.sync_copy(rk_v, ranks_hbm.at[pl.ds(w * LPW * 16, LPW * 16)])


def _k4_body(ranks_hbm, ap_hbm, rksp_v, rk_v, p_v, out_v):
    w = _wid()

    @pl.when(w == 0)
    def _():
        pltpu.sync_copy(ranks_hbm, rksp_v)
        lanes = lax.iota(jnp.int32, 16)
        lane0 = lanes == 0

        def flat_body(c, carry):
            rv = rksp_v[pl.ds(c * 16, 16)]
            plsc.store_scatter(rk_v, [jnp.full((16,), c, jnp.int32)], rv,
                               mask=lane0)
            return carry

        lax.fori_loop(0, ML, flat_body, 0)

        def pa(c, carry):
            cv = jnp.full((16,), c, jnp.int32)
            rcv = plsc.load_gather(rk_v, [cv])
            rc = jnp.max(rcv)

            def ib(i, acc):
                rv = rk_v[pl.ds(i * 16, 16)]
                return acc + (rv < rcv).astype(jnp.int32)

            acc = lax.fori_loop(0, MLV, ib, jnp.zeros((16,), jnp.int32))
            pos = jnp.sum(acc)
            valid = rc < BIG
            pvec = (jnp.full((16,), pos, jnp.float32) + 1.0) / \
                   (rcv.astype(jnp.float32) + 1.0)
            pvec = jnp.where(valid, pvec, -1.0)
            plsc.store_scatter(p_v, [cv], pvec, mask=lane0)
            return carry

        lax.fori_loop(0, ML, pa, 0)

        def pb(c, ap):
            cv = jnp.full((16,), c, jnp.int32)
            rcv = plsc.load_gather(rk_v, [cv])
            rc = jnp.max(rcv)

            def ib(i, pm):
                rv = rk_v[pl.ds(i * 16, 16)]
                pv = p_v[pl.ds(i * 16, 16)]
                return jnp.maximum(pm, jnp.where(rv >= rcv, pv, -1.0))

            pmv = lax.fori_loop(0, MLV, ib, jnp.full((16,), -1.0, jnp.float32))
            pm = jnp.max(pmv)
            use = jnp.logical_and(rc < BIG, rc >= 1)
            return ap + jnp.where(use, pm, 0.0)

        ap = lax.fori_loop(0, ML, pb, jnp.float32(0.0))
        out_v[...] = jnp.full((16,), ap * jnp.float32(1.0 / NLBL),
                              jnp.float32)
        pltpu.sync_copy(out_v, ap_hbm)


@functools.lru_cache(maxsize=1)
def _build():
    mesh = plsc.VectorSubcoreMesh(
        core_axis_name="c", subcore_axis_name="s",
        num_cores=NCORES, num_subcores=NSUB)
    k1 = pl.kernel(
        _k1_body, mesh=mesh,
        compiler_params=pltpu.CompilerParams(needs_layout_passes=False),
        out_type=[jax.ShapeDtypeStruct((ML * CAP,), jnp.int32),
                  jax.ShapeDtypeStruct((ML * 16,), jnp.int32)],
        scratch_types=[pltpu.VMEM((N,), jnp.float32),
                       pltpu.VMEM((N,), jnp.float32),
                       pltpu.VMEM((LPW * 16,), jnp.float32),
                       pltpu.VMEM((LPW * 16,), jnp.float32),
                       pltpu.VMEM((CAP,), jnp.int32),
                       pltpu.VMEM((LPW * 16,), jnp.int32)])
    k2 = pl.kernel(
        _k2_body, mesh=mesh,
        compiler_params=pltpu.CompilerParams(needs_layout_passes=False),
        out_type=[jax.ShapeDtypeStruct((ML * 16,), jnp.int32),
                  jax.ShapeDtypeStruct((ML * 16,), jnp.float32)],
        scratch_types=[pltpu.VMEM((ML * CAP,), jnp.int32),
                       pltpu.VMEM((ML * 16,), jnp.int32),
                       pltpu.VMEM((N,), jnp.float32),
                       pltpu.VMEM((N,), jnp.int32),
                       pltpu.VMEM((ML * 16,), jnp.int32),
                       pltpu.VMEM((ML * 16,), jnp.float32)])
    k3 = pl.kernel(
        _k3_body, mesh=mesh,
        compiler_params=pltpu.CompilerParams(needs_layout_passes=False),
        out_type=jax.ShapeDtypeStruct((ML * 16,), jnp.int32),
        scratch_types=[pltpu.VMEM((N,), jnp.float32),
                       pltpu.VMEM((LPW * 16,), jnp.int32),
                       pltpu.VMEM((LPW * 16,), jnp.float32),
                       pltpu.VMEM((LPW * 16,), jnp.int32)])
    k4 = pl.kernel(
        _k4_body, mesh=mesh,
        compiler_params=pltpu.CompilerParams(needs_layout_passes=False),
        out_type=jax.ShapeDtypeStruct((16,), jnp.float32),
        scratch_types=[pltpu.VMEM((ML * 16,), jnp.int32),
                       pltpu.VMEM((ML,), jnp.int32),
                       pltpu.VMEM((ML,), jnp.float32),
                       pltpu.VMEM((16,), jnp.float32)])
    return k1, k2, k3, k4


def kernel(scores, segments, labels):
    smin = segments[:, 0]
    smax = segments[:, 1]
    lmin = jnp.concatenate(
        [labels[:, 0], jnp.full((ML - NLBL,), -2.0, jnp.float32)])
    lmax = jnp.concatenate(
        [labels[:, 1], jnp.full((ML - NLBL,), -1.0, jnp.float32)])
    lmin16 = jnp.broadcast_to(lmin[:, None], (ML, 16)).reshape(ML * 16)
    lmax16 = jnp.broadcast_to(lmax[:, None], (ML, 16)).reshape(ML * 16)

    k1, k2, k3, k4 = _build()
    lists, counts = k1(smin, smax, lmin16, lmax16)
    chosen, svals = k2(lists, counts, scores)
    ranks = k3(scores, chosen, svals)
    ap16 = k4(ranks)
    return ap16[0]


# speedup vs baseline: 5.8679x; 1.1457x over previous
import functools

import jax
import jax.numpy as jnp
from jax import lax
from jax.experimental import pallas as pl
from jax.experimental.pallas import tpu as pltpu
from jax.experimental.pallas import tpu_sc as plsc

N = 20000
NV = N // 16
NLBL = 200
NCORES = 2
NSUB = 16
NW = NCORES * NSUB
LPW = 7
ML = NW * LPW
MLV = ML // 16
CAP = 224
K = 200
BIG = 1 << 30

def _wid():
    return lax.axis_index("s") * NCORES + lax.axis_index("c")


def _k1_body(smin_hbm, smax_hbm, lmin_hbm, lmax_hbm, lists_hbm, counts_hbm,
             smin_v, smax_v, lmin_v, lmax_v, list_v, cnt_v):
    w = _wid()
    base_l = w * LPW
    pltpu.sync_copy(smin_hbm, smin_v)
    pltpu.sync_copy(smax_hbm, smax_v)
    pltpu.sync_copy(lmin_hbm.at[pl.ds(base_l * 16, LPW * 16)], lmin_v)
    pltpu.sync_copy(lmax_hbm.at[pl.ds(base_l * 16, LPW * 16)], lmax_v)
    lanes = lax.iota(jnp.int32, 16)
    bmin = [lmin_v[pl.ds(j * 16, 16)] for j in range(LPW)]
    bmax = [lmax_v[pl.ds(j * 16, 16)] for j in range(LPW)]
    blen = [bmax[j] - bmin[j] for j in range(LPW)]
    kvec = jnp.full((16,), K, jnp.int32)

    def body(i, offs):
        b = i * 16
        sm = smin_v[pl.ds(b, 16)]
        sx = smax_v[pl.ds(b, 16)]
        la = sx - sm
        iv = lanes + b
        new_offs = []
        for j in range(LPW):
            inter = jnp.maximum(
                jnp.minimum(sx, bmax[j]) - jnp.maximum(sm, bmin[j]), 0.0)
            m = (inter + inter) > (la + blen[j] - inter)
            off = offs[j]

            @pl.when(jnp.any(m))
            def _(m=m, off=off, iv=iv, j=j):
                mstore = jnp.logical_and(m, off < kvec)
                mi = mstore.astype(jnp.int32)
                dest = jnp.minimum(off, kvec) + plsc.cumsum(mi) - mi
                plsc.store_scatter(list_v, [dest + (j * CAP)], iv,
                                   mask=mstore)

            new_offs.append(off + plsc.all_reduce_population_count(m))
        return tuple(new_offs)

    offs = lax.fori_loop(
        0, NV, body, tuple(jnp.zeros((16,), jnp.int32) for _ in range(LPW)))
    for j in range(LPW):
        cnt_v[pl.ds(j * 16, 16)] = offs[j]
    pltpu.sync_copy(list_v, lists_hbm.at[pl.ds(base_l * CAP, LPW * CAP)])
    pltpu.sync_copy(cnt_v, counts_hbm.at[pl.ds(w * LPW * 16, LPW * 16)])


def _k2_body(lists_hbm, counts_hbm, scores_hbm, chosen_hbm, svals_hbm,
             lists_v, counts_v, scores_v, taken_v, chosen_v, svals_v):
    w = _wid()

    @pl.when(w == 0)
    def _():
        pltpu.sync_copy(lists_hbm, lists_v)
        pltpu.sync_copy(counts_hbm, counts_v)
        pltpu.sync_copy(scores_hbm, scores_v)
        lanes = lax.iota(jnp.int32, 16)
        zeros = jnp.zeros((16,), jnp.int32)
        ones = jnp.ones((16,), jnp.int32)

        def zbody(i, carry):
            taken_v[pl.ds(i * 16, 16)] = zeros
            return carry

        lax.fori_loop(0, NV, zbody, 0)

        def lbody(c, carry):
            cnt = jnp.max(counts_v[pl.ds(c * 16, 16)])
            cntcap = jnp.minimum(cnt, K)

            def body(i, ch):
                def scan_vreg():
                    candv = lists_v[pl.ds(c * CAP + i * 16, 16)]
                    validm = (lanes + i * 16) < cntcap
                    csafe = jnp.where(validm, candv, 0)
                    tk = plsc.load_gather(taken_v, [csafe])
                    avail = jnp.logical_and(validm, tk == 0)
                    chn = jnp.min(jnp.where(avail, candv, BIG))
                    return jnp.where(chn < BIG, chn, -1)

                pred = jnp.logical_and(ch < 0, i * 16 < cntcap)
                return lax.cond(pred, scan_vreg, lambda: ch)

            ch = lax.fori_loop(0, (K + 15) // 16, body, jnp.int32(-1))
            has = ch >= 0
            chv = jnp.full((16,), ch, jnp.int32)
            csafe = jnp.maximum(chv, 0)
            plsc.store_scatter(taken_v, [csafe], ones,
                               mask=jnp.logical_and(lanes == 0, has))
            sv = plsc.load_gather(scores_v, [csafe])
            chosen_v[pl.ds(c * 16, 16)] = chv
            svals_v[pl.ds(c * 16, 16)] = jnp.where(has, sv, 0.0)
            return carry

        lax.fori_loop(0, ML, lbody, 0)
        pltpu.sync_copy(chosen_v, chosen_hbm)
        pltpu.sync_copy(svals_v, svals_hbm)


def _k3_body(scores_hbm, chosen_hbm, svals_hbm, ranks_hbm,
             scores_v, ch_v, sv_v, rk_v):
    w = _wid()
    pltpu.sync_copy(scores_hbm, scores_v)
    pltpu.sync_copy(chosen_hbm.at[pl.ds(w * LPW * 16, LPW * 16)], ch_v)
    pltpu.sync_copy(svals_hbm.at[pl.ds(w * LPW * 16, LPW * 16)], sv_v)
    lanes = lax.iota(jnp.int32, 16)
    jv = [ch_v[pl.ds(j * 16, 16)] for j in range(LPW)]
    sv = [sv_v[pl.ds(j * 16, 16)] for j in range(LPW)]

    def body(i, accs):
        b = i * 16
        sc = scores_v[pl.ds(b, 16)]
        idxv = lanes + b
        return tuple(
            accs[j] + jnp.logical_or(
                sc > sv[j],
                jnp.logical_and(sc == sv[j], idxv < jv[j])
            ).astype(jnp.int32)
            for j in range(LPW))

    accs = lax.fori_loop(
        0, NV, body, tuple(jnp.zeros((16,), jnp.int32) for _ in range(LPW)))
    for j in range(LPW):
        rank = jnp.where(jnp.max(jv[j]) >= 0, jnp.sum(accs[j]), BIG)
        rk_v[pl.ds(j * 16, 16)] = jnp.full((16,), rank, jnp.int32)
    pltpu.sync_copy(rk_v, ranks_hbm.at[pl.ds(w * LPW * 16, LPW * 16)])


def _k4_body(ranks_hbm, ap_hbm, rksp_v, rk_v, p_v, out_v):
    w = _wid()

    @pl.when(w == 0)
    def _():
        pltpu.sync_copy(ranks_hbm, rksp_v)
        lanes = lax.iota(jnp.int32, 16)
        lane0 = lanes == 0

        def flat_body(c, carry):
            rv = rksp_v[pl.ds(c * 16, 16)]
            plsc.store_scatter(rk_v, [jnp.full((16,), c, jnp.int32)], rv,
                               mask=lane0)
            return carry

        lax.fori_loop(0, ML, flat_body, 0)

        def pa(c, carry):
            cv = jnp.full((16,), c, jnp.int32)
            rcv = plsc.load_gather(rk_v, [cv])
            rc = jnp.max(rcv)

            def ib(i, acc):
                rv = rk_v[pl.ds(i * 16, 16)]
                return acc + (rv < rcv).astype(jnp.int32)

            acc = lax.fori_loop(0, MLV, ib, jnp.zeros((16,), jnp.int32))
            pos = jnp.sum(acc)
            valid = rc < BIG
            pvec = (jnp.full((16,), pos, jnp.float32) + 1.0) / \
                   (rcv.astype(jnp.float32) + 1.0)
            pvec = jnp.where(valid, pvec, -1.0)
            plsc.store_scatter(p_v, [cv], pvec, mask=lane0)
            return carry

        lax.fori_loop(0, ML, pa, 0)

        def pb(c, ap):
            cv = jnp.full((16,), c, jnp.int32)
            rcv = plsc.load_gather(rk_v, [cv])
            rc = jnp.max(rcv)

            def ib(i, pm):
                rv = rk_v[pl.ds(i * 16, 16)]
                pv = p_v[pl.ds(i * 16, 16)]
                return jnp.maximum(pm, jnp.where(rv >= rcv, pv, -1.0))

            pmv = lax.fori_loop(0, MLV, ib, jnp.full((16,), -1.0, jnp.float32))
            pm = jnp.max(pmv)
            use = jnp.logical_and(rc < BIG, rc >= 1)
            return ap + jnp.where(use, pm, 0.0)

        ap = lax.fori_loop(0, ML, pb, jnp.float32(0.0))
        out_v[...] = jnp.full((16,), ap * jnp.float32(1.0 / NLBL),
                              jnp.float32)
        pltpu.sync_copy(out_v, ap_hbm)


@functools.lru_cache(maxsize=1)
def _build():
    mesh = plsc.VectorSubcoreMesh(
        core_axis_name="c", subcore_axis_name="s",
        num_cores=NCORES, num_subcores=NSUB)
    k1 = pl.kernel(
        _k1_body, mesh=mesh,
        compiler_params=pltpu.CompilerParams(needs_layout_passes=False),
        out_type=[jax.ShapeDtypeStruct((ML * CAP,), jnp.int32),
                  jax.ShapeDtypeStruct((ML * 16,), jnp.int32)],
        scratch_types=[pltpu.VMEM((N,), jnp.float32),
                       pltpu.VMEM((N,), jnp.float32),
                       pltpu.VMEM((LPW * 16,), jnp.float32),
                       pltpu.VMEM((LPW * 16,), jnp.float32),
                       pltpu.VMEM((LPW * CAP,), jnp.int32),
                       pltpu.VMEM((LPW * 16,), jnp.int32)])
    k2 = pl.kernel(
        _k2_body, mesh=mesh,
        compiler_params=pltpu.CompilerParams(needs_layout_passes=False),
        out_type=[jax.ShapeDtypeStruct((ML * 16,), jnp.int32),
                  jax.ShapeDtypeStruct((ML * 16,), jnp.float32)],
        scratch_types=[pltpu.VMEM((ML * CAP,), jnp.int32),
                       pltpu.VMEM((ML * 16,), jnp.int32),
                       pltpu.VMEM((N,), jnp.float32),
                       pltpu.VMEM((N,), jnp.int32),
                       pltpu.VMEM((ML * 16,), jnp.int32),
                       pltpu.VMEM((ML * 16,), jnp.float32)])
    k3 = pl.kernel(
        _k3_body, mesh=mesh,
        compiler_params=pltpu.CompilerParams(needs_layout_passes=False),
        out_type=jax.ShapeDtypeStruct((ML * 16,), jnp.int32),
        scratch_types=[pltpu.VMEM((N,), jnp.float32),
                       pltpu.VMEM((LPW * 16,), jnp.int32),
                       pltpu.VMEM((LPW * 16,), jnp.float32),
                       pltpu.VMEM((LPW * 16,), jnp.int32)])
    k4 = pl.kernel(
        _k4_body, mesh=mesh,
        compiler_params=pltpu.CompilerParams(needs_layout_passes=False),
        out_type=jax.ShapeDtypeStruct((16,), jnp.float32),
        scratch_types=[pltpu.VMEM((ML * 16,), jnp.int32),
                       pltpu.VMEM((ML,), jnp.int32),
                       pltpu.VMEM((ML,), jnp.float32),
                       pltpu.VMEM((16,), jnp.float32)])
    return k1, k2, k3, k4


def kernel(scores, segments, labels):
    smin = segments[:, 0]
    smax = segments[:, 1]
    lmin = jnp.concatenate(
        [labels[:, 0], jnp.full((ML - NLBL,), -2.0, jnp.float32)])
    lmax = jnp.concatenate(
        [labels[:, 1], jnp.full((ML - NLBL,), -1.0, jnp.float32)])
    lmin16 = jnp.broadcast_to(lmin[:, None], (ML, 16)).reshape(ML * 16)
    lmax16 = jnp.broadcast_to(lmax[:, None], (ML, 16)).reshape(ML * 16)

    k1, k2, k3, k4 = _build()
    lists, counts = k1(smin, smax, lmin16, lmax16)
    chosen, svals = k2(lists, counts, scores)
    ranks = k3(scores, chosen, svals)
    ap16 = k4(ranks)
    return ap16[0]


# speedup vs baseline: 10.2784x; 1.7516x over previous
import functools

import jax
import jax.numpy as jnp
from jax import lax
from jax.experimental import pallas as pl
from jax.experimental.pallas import tpu as pltpu
from jax.experimental.pallas import tpu_sc as plsc

N = 20000
NV = N // 16
NLBL = 200
NCORES = 2
NSUB = 16
NW = NCORES * NSUB
LPW = 7
ML = NW * LPW
MLV = ML // 16
CAP = 224
K = 200
BIG = 1 << 30
NSLOT = NW * 8
NSV = NSLOT // 16

def _wid():
    return lax.axis_index("s") * NCORES + lax.axis_index("c")


def _k1_body(smin_hbm, smax_hbm, lmin_hbm, lmax_hbm, lists_hbm, counts_hbm,
             smin_v, smax_v, lmin_v, lmax_v, list_v, cnt_v):
    w = _wid()
    base_l = w * LPW
    pltpu.sync_copy(smin_hbm, smin_v)
    pltpu.sync_copy(smax_hbm, smax_v)
    pltpu.sync_copy(lmin_hbm.at[pl.ds(base_l * 16, LPW * 16)], lmin_v)
    pltpu.sync_copy(lmax_hbm.at[pl.ds(base_l * 16, LPW * 16)], lmax_v)
    lanes = lax.iota(jnp.int32, 16)
    bmin = [lmin_v[pl.ds(j * 16, 16)] for j in range(LPW)]
    bmax = [lmax_v[pl.ds(j * 16, 16)] for j in range(LPW)]
    blen = [bmax[j] - bmin[j] for j in range(LPW)]
    kvec = jnp.full((16,), K, jnp.int32)

    def body(i, offs):
        b = i * 16
        sm = smin_v[pl.ds(b, 16)]
        sx = smax_v[pl.ds(b, 16)]
        la = sx - sm
        iv = lanes + b
        new_offs = []
        for j in range(LPW):
            inter = jnp.maximum(
                jnp.minimum(sx, bmax[j]) - jnp.maximum(sm, bmin[j]), 0.0)
            m = (inter * 3.0) > (la + blen[j])
            off = offs[j]
            mstore = jnp.logical_and(m, off < kvec)
            mi = mstore.astype(jnp.int32)
            dest = jnp.minimum(off, kvec) + plsc.cumsum(mi) - mi
            plsc.store_scatter(list_v, [dest + (j * CAP)], iv, mask=mstore)
            new_offs.append(off + plsc.all_reduce_population_count(m))
        return tuple(new_offs)

    offs = lax.fori_loop(
        0, NV, body, tuple(jnp.zeros((16,), jnp.int32) for _ in range(LPW)))
    for j in range(LPW):
        cnt_v[pl.ds(j * 16, 16)] = offs[j]
    pltpu.sync_copy(list_v, lists_hbm.at[pl.ds(base_l * CAP, LPW * CAP)])
    pltpu.sync_copy(cnt_v, counts_hbm.at[pl.ds(w * LPW * 16, LPW * 16)])


def _k2_body(lists_hbm, counts_hbm, scores_hbm, chosen_hbm, svals_hbm,
             lists_v, counts_v, scores_v, taken_v, chosen_v, svals_v):
    w = _wid()

    @pl.when(w == 0)
    def _():
        pltpu.sync_copy(lists_hbm, lists_v)
        pltpu.sync_copy(counts_hbm, counts_v)
        pltpu.sync_copy(scores_hbm, scores_v)
        lanes = lax.iota(jnp.int32, 16)
        zeros = jnp.zeros((16,), jnp.int32)
        ones = jnp.ones((16,), jnp.int32)

        def zbody(i, carry):
            taken_v[pl.ds(i * 16, 16)] = zeros
            return carry

        lax.fori_loop(0, NV, zbody, 0)

        def lbody(c, carry):
            cnt = jnp.max(counts_v[pl.ds(c * 16, 16)])
            cntcap = jnp.minimum(cnt, K)

            def scan_vreg(i):
                candv = lists_v[pl.ds(c * CAP + i * 16, 16)]
                validm = (lanes + i * 16) < cntcap
                csafe = jnp.where(validm, candv, 0)
                tk = plsc.load_gather(taken_v, [csafe])
                avail = jnp.logical_and(validm, tk == 0)
                chn = jnp.min(jnp.where(avail, candv, BIG))
                return jnp.where(chn < BIG, chn, -1)

            ch = scan_vreg(jnp.int32(0))

            def tail():
                def body(i, ch2):
                    chn = scan_vreg(i)
                    return jnp.where(ch2 >= 0, ch2, chn)
                return lax.fori_loop(1, (K + 15) // 16, body, jnp.int32(-1))

            ch = lax.cond(jnp.logical_and(ch < 0, cntcap > 16),
                          tail, lambda: ch)
            has = ch >= 0
            chv = jnp.full((16,), ch, jnp.int32)
            csafe = jnp.maximum(chv, 0)
            plsc.store_scatter(taken_v, [csafe], ones,
                               mask=jnp.logical_and(lanes == 0, has))
            sv = plsc.load_gather(scores_v, [csafe])
            chosen_v[pl.ds(c * 16, 16)] = chv
            svals_v[pl.ds(c * 16, 16)] = jnp.where(has, sv, 0.0)
            return carry

        lax.fori_loop(0, ML, lbody, 0)
        pltpu.sync_copy(chosen_v, chosen_hbm)
        pltpu.sync_copy(svals_v, svals_hbm)


def _k3_body(scores_hbm, chosen_hbm, svals_hbm, ranks_hbm,
             scores_v, ch_v, sv_v, rk_v):
    w = _wid()
    pltpu.sync_copy(scores_hbm, scores_v)
    pltpu.sync_copy(chosen_hbm.at[pl.ds(w * LPW * 16, LPW * 16)], ch_v)
    pltpu.sync_copy(svals_hbm.at[pl.ds(w * LPW * 16, LPW * 16)], sv_v)
    lanes = lax.iota(jnp.int32, 16)
    jv = [ch_v[pl.ds(j * 16, 16)] for j in range(LPW)]
    sv = [sv_v[pl.ds(j * 16, 16)] for j in range(LPW)]

    def body(i, accs):
        b = i * 16
        sc = scores_v[pl.ds(b, 16)]
        idxv = lanes + b
        return tuple(
            accs[j] + jnp.logical_or(
                sc > sv[j],
                jnp.logical_and(sc == sv[j], idxv < jv[j])
            ).astype(jnp.int32)
            for j in range(LPW))

    accs = lax.fori_loop(
        0, NV, body, tuple(jnp.zeros((16,), jnp.int32) for _ in range(LPW)))
    lane0 = lanes == 0
    for j in range(LPW + 1):
        if j < LPW:
            rank = jnp.where(jnp.max(jv[j]) >= 0, jnp.sum(accs[j]), BIG)
        else:
            rank = jnp.int32(BIG)
        plsc.store_scatter(rk_v, [jnp.full((16,), j, jnp.int32)],
                           jnp.full((16,), rank, jnp.int32), mask=lane0)
    pltpu.sync_copy(rk_v, ranks_hbm.at[pl.ds(w * 8, 8)])


def _k4_body(ranks_hbm, ap_hbm, rk_v, p_v, out_v):
    w = _wid()

    @pl.when(w == 0)
    def _():
        pltpu.sync_copy(ranks_hbm, rk_v)
        lanes = lax.iota(jnp.int32, 16)
        lane0 = lanes == 0

        def pa(c, carry):
            cv = jnp.full((16,), c, jnp.int32)
            rcv = plsc.load_gather(rk_v, [cv])
            rc = jnp.max(rcv)

            def ib(i, acc):
                rv = rk_v[pl.ds(i * 16, 16)]
                return acc + (rv < rcv).astype(jnp.int32)

            acc = lax.fori_loop(0, NSV, ib, jnp.zeros((16,), jnp.int32))
            pos = jnp.sum(acc)
            valid = rc < BIG
            pvec = (jnp.full((16,), pos, jnp.float32) + 1.0) / \
                   (rcv.astype(jnp.float32) + 1.0)
            pvec = jnp.where(valid, pvec, -1.0)
            plsc.store_scatter(p_v, [cv], pvec, mask=lane0)
            return carry

        lax.fori_loop(0, NSLOT, pa, 0)

        def pb(c, ap):
            cv = jnp.full((16,), c, jnp.int32)
            rcv = plsc.load_gather(rk_v, [cv])
            rc = jnp.max(rcv)

            def ib(i, pm):
                rv = rk_v[pl.ds(i * 16, 16)]
                pv = p_v[pl.ds(i * 16, 16)]
                return jnp.maximum(pm, jnp.where(rv >= rcv, pv, -1.0))

            pmv = lax.fori_loop(0, NSV, ib, jnp.full((16,), -1.0, jnp.float32))
            pm = jnp.max(pmv)
            use = jnp.logical_and(rc < BIG, rc >= 1)
            return ap + jnp.where(use, pm, 0.0)

        ap = lax.fori_loop(0, NSLOT, pb, jnp.float32(0.0))
        out_v[...] = jnp.full((16,), ap * jnp.float32(1.0 / NLBL),
                              jnp.float32)
        pltpu.sync_copy(out_v, ap_hbm)


@functools.lru_cache(maxsize=1)
def _build():
    mesh = plsc.VectorSubcoreMesh(
        core_axis_name="c", subcore_axis_name="s",
        num_cores=NCORES, num_subcores=NSUB)
    k1 = pl.kernel(
        _k1_body, mesh=mesh,
        compiler_params=pltpu.CompilerParams(needs_layout_passes=False),
        out_type=[jax.ShapeDtypeStruct((ML * CAP,), jnp.int32),
                  jax.ShapeDtypeStruct((ML * 16,), jnp.int32)],
        scratch_types=[pltpu.VMEM((N,), jnp.float32),
                       pltpu.VMEM((N,), jnp.float32),
                       pltpu.VMEM((LPW * 16,), jnp.float32),
                       pltpu.VMEM((LPW * 16,), jnp.float32),
                       pltpu.VMEM((LPW * CAP,), jnp.int32),
                       pltpu.VMEM((LPW * 16,), jnp.int32)])
    k2 = pl.kernel(
        _k2_body, mesh=mesh,
        compiler_params=pltpu.CompilerParams(needs_layout_passes=False),
        out_type=[jax.ShapeDtypeStruct((ML * 16,), jnp.int32),
                  jax.ShapeDtypeStruct((ML * 16,), jnp.float32)],
        scratch_types=[pltpu.VMEM((ML * CAP,), jnp.int32),
                       pltpu.VMEM((ML * 16,), jnp.int32),
                       pltpu.VMEM((N,), jnp.float32),
                       pltpu.VMEM((N,), jnp.int32),
                       pltpu.VMEM((ML * 16,), jnp.int32),
                       pltpu.VMEM((ML * 16,), jnp.float32)])
    k3 = pl.kernel(
        _k3_body, mesh=mesh,
        compiler_params=pltpu.CompilerParams(needs_layout_passes=False),
        out_type=jax.ShapeDtypeStruct((NSLOT,), jnp.int32),
        scratch_types=[pltpu.VMEM((N,), jnp.float32),
                       pltpu.VMEM((LPW * 16,), jnp.int32),
                       pltpu.VMEM((LPW * 16,), jnp.float32),
                       pltpu.VMEM((8,), jnp.int32)])
    k4 = pl.kernel(
        _k4_body, mesh=mesh,
        compiler_params=pltpu.CompilerParams(needs_layout_passes=False),
        out_type=jax.ShapeDtypeStruct((16,), jnp.float32),
        scratch_types=[pltpu.VMEM((NSLOT,), jnp.int32),
                       pltpu.VMEM((NSLOT,), jnp.float32),
                       pltpu.VMEM((16,), jnp.float32)])
    return k1, k2, k3, k4


def kernel(scores, segments, labels):
    smin = segments[:, 0]
    smax = segments[:, 1]
    lmin = jnp.concatenate(
        [labels[:, 0], jnp.full((ML - NLBL,), -2.0, jnp.float32)])
    lmax = jnp.concatenate(
        [labels[:, 1], jnp.full((ML - NLBL,), -1.0, jnp.float32)])
    lmin16 = jnp.broadcast_to(lmin[:, None], (ML, 16)).reshape(ML * 16)
    lmax16 = jnp.broadcast_to(lmax[:, None], (ML, 16)).reshape(ML * 16)

    k1, k2, k3, k4 = _build()
    lists, counts = k1(smin, smax, lmin16, lmax16)
    chosen, svals = k2(lists, counts, scores)
    ranks = k3(scores, chosen, svals)
    ap16 = k4(ranks)
    return ap16[0]


# speedup vs baseline: 10.6637x; 1.0375x over previous
import functools

import jax
import jax.numpy as jnp
from jax import lax
from jax.experimental import pallas as pl
from jax.experimental.pallas import tpu as pltpu
from jax.experimental.pallas import tpu_sc as plsc

N = 20000
NV = N // 16
NLBL = 200
NCORES = 2
NSUB = 16
NW = NCORES * NSUB
LPW = 7
ML = NW * LPW
MLV = ML // 16
CAP = 224
K = 200
BIG = 1 << 30
NSLOT = NW * 8
NSV = NSLOT // 16

def _wid():
    return lax.axis_index("s") * NCORES + lax.axis_index("c")


def _k1_body(smin_hbm, smax_hbm, lmin_hbm, lmax_hbm, lists_hbm,
             smin_v, smax_v, lmin_v, lmax_v, list_v):
    w = _wid()
    base_l = w * LPW
    pltpu.sync_copy(smin_hbm, smin_v)
    pltpu.sync_copy(smax_hbm, smax_v)
    pltpu.sync_copy(lmin_hbm.at[pl.ds(base_l * 16, LPW * 16)], lmin_v)
    pltpu.sync_copy(lmax_hbm.at[pl.ds(base_l * 16, LPW * 16)], lmax_v)
    lanes = lax.iota(jnp.int32, 16)

    bigv = jnp.full((16,), BIG, jnp.int32)

    def initb(i, carry):
        list_v[pl.ds(i * 16, 16)] = bigv
        return carry

    lax.fori_loop(0, LPW * CAP // 16, initb, 0)
    bmin = [lmin_v[pl.ds(j * 16, 16)] for j in range(LPW)]
    bmax = [lmax_v[pl.ds(j * 16, 16)] for j in range(LPW)]
    blen = [bmax[j] - bmin[j] for j in range(LPW)]
    kvec = jnp.full((16,), K, jnp.int32)

    def body(i, offs):
        b = i * 16
        sm = smin_v[pl.ds(b, 16)]
        sx = smax_v[pl.ds(b, 16)]
        la = sx - sm
        iv = lanes + b
        new_offs = []
        for j in range(LPW):
            inter = jnp.maximum(
                jnp.minimum(sx, bmax[j]) - jnp.maximum(sm, bmin[j]), 0.0)
            m = (inter * 3.0) > (la + blen[j])
            off = offs[j]
            mstore = jnp.logical_and(m, off < kvec)
            mi = mstore.astype(jnp.int32)
            dest = jnp.minimum(off, kvec) + plsc.cumsum(mi) - mi
            plsc.store_scatter(list_v, [dest + (j * CAP)], iv, mask=mstore)
            new_offs.append(off + plsc.all_reduce_population_count(m))
        return tuple(new_offs)

    lax.fori_loop(
        0, NV, body, tuple(jnp.zeros((16,), jnp.int32) for _ in range(LPW)))
    pltpu.sync_copy(list_v, lists_hbm.at[pl.ds(base_l * CAP, LPW * CAP)])


def _k2_body(lists_hbm, scores_hbm, chosen_hbm, svals_hbm,
             lists_v, scores_v, taken_v, chosen_v, svals_v):
    w = _wid()

    @pl.when(w == 0)
    def _():
        pltpu.sync_copy(lists_hbm, lists_v)
        pltpu.sync_copy(scores_hbm, scores_v)
        lanes = lax.iota(jnp.int32, 16)
        zeros = jnp.zeros((16,), jnp.int32)
        ones = jnp.ones((16,), jnp.int32)
        nvec = jnp.full((16,), N, jnp.int32)

        def zbody(i, carry):
            taken_v[pl.ds(i * 16, 16)] = zeros
            return carry

        lax.fori_loop(0, NV + 1, zbody, 0)

        def lbody(c, carry):
            def scan_vreg(i):
                candv = lists_v[pl.ds(c * CAP + i * 16, 16)]
                csafe = jnp.minimum(candv, nvec)
                tk = plsc.load_gather(taken_v, [csafe])
                avail = jnp.logical_and(tk == 0, candv < nvec)
                chn = jnp.min(jnp.where(avail, candv, BIG))
                return jnp.where(chn < BIG, chn, -1)

            candv0 = lists_v[pl.ds(c * CAP, 16)]
            csafe0 = jnp.minimum(candv0, nvec)
            tk0 = plsc.load_gather(taken_v, [csafe0])
            avail0 = jnp.logical_and(tk0 == 0, candv0 < nvec)
            chn0 = jnp.min(jnp.where(avail0, candv0, BIG))
            ch = jnp.where(chn0 < BIG, chn0, -1)
            all16 = jnp.all(candv0 < nvec)

            def tail():
                def body(i, ch2):
                    chn = scan_vreg(i)
                    return jnp.where(ch2 >= 0, ch2, chn)
                return lax.fori_loop(1, (K + 15) // 16, body, jnp.int32(-1))

            ch = lax.cond(jnp.logical_and(ch < 0, all16), tail, lambda: ch)
            has = ch >= 0
            chv = jnp.full((16,), ch, jnp.int32)
            csafe = jnp.maximum(chv, 0)
            plsc.store_scatter(taken_v, [csafe], ones,
                               mask=jnp.logical_and(lanes == 0, has))
            sv = plsc.load_gather(scores_v, [csafe])
            chosen_v[pl.ds(c * 16, 16)] = chv
            svals_v[pl.ds(c * 16, 16)] = jnp.where(has, sv, 0.0)
            return carry

        lax.fori_loop(0, ML, lbody, 0)
        pltpu.sync_copy(chosen_v, chosen_hbm)
        pltpu.sync_copy(svals_v, svals_hbm)


def _k3_body(scores_hbm, chosen_hbm, svals_hbm, ranks_hbm,
             scores_v, ch_v, sv_v, rk_v):
    w = _wid()
    pltpu.sync_copy(scores_hbm, scores_v)
    pltpu.sync_copy(chosen_hbm.at[pl.ds(w * LPW * 16, LPW * 16)], ch_v)
    pltpu.sync_copy(svals_hbm.at[pl.ds(w * LPW * 16, LPW * 16)], sv_v)
    lanes = lax.iota(jnp.int32, 16)
    jv = [ch_v[pl.ds(j * 16, 16)] for j in range(LPW)]
    sv = [sv_v[pl.ds(j * 16, 16)] for j in range(LPW)]

    def body(i, accs):
        b = i * 16
        sc = scores_v[pl.ds(b, 16)]
        idxv = lanes + b
        return tuple(
            accs[j] + jnp.logical_or(
                sc > sv[j],
                jnp.logical_and(sc == sv[j], idxv < jv[j])
            ).astype(jnp.int32)
            for j in range(LPW))

    accs = lax.fori_loop(
        0, NV, body, tuple(jnp.zeros((16,), jnp.int32) for _ in range(LPW)))
    lane0 = lanes == 0
    for j in range(LPW + 1):
        if j < LPW:
            rank = jnp.where(jnp.max(jv[j]) >= 0, jnp.sum(accs[j]), BIG)
        else:
            rank = jnp.int32(BIG)
        plsc.store_scatter(rk_v, [jnp.full((16,), j, jnp.int32)],
                           jnp.full((16,), rank, jnp.int32), mask=lane0)
    pltpu.sync_copy(rk_v, ranks_hbm.at[pl.ds(w * 8, 8)])


def _k4_body(ranks_hbm, ap_hbm, rk_v, p_v, out_v):
    w = _wid()

    @pl.when(w == 0)
    def _():
        pltpu.sync_copy(ranks_hbm, rk_v)
        lanes = lax.iota(jnp.int32, 16)
        lane0 = lanes == 0

        def pa(c, carry):
            cv = jnp.full((16,), c, jnp.int32)
            rcv = plsc.load_gather(rk_v, [cv])
            rc = jnp.max(rcv)

            def ib(i, acc):
                rv = rk_v[pl.ds(i * 16, 16)]
                return acc + (rv < rcv).astype(jnp.int32)

            acc = lax.fori_loop(0, NSV, ib, jnp.zeros((16,), jnp.int32))
            pos = jnp.sum(acc)
            valid = rc < BIG
            pvec = (jnp.full((16,), pos, jnp.float32) + 1.0) / \
                   (rcv.astype(jnp.float32) + 1.0)
            pvec = jnp.where(valid, pvec, -1.0)
            plsc.store_scatter(p_v, [cv], pvec, mask=lane0)
            return carry

        lax.fori_loop(0, NSLOT, pa, 0)

        def pb(c, ap):
            cv = jnp.full((16,), c, jnp.int32)
            rcv = plsc.load_gather(rk_v, [cv])
            rc = jnp.max(rcv)

            def ib(i, pm):
                rv = rk_v[pl.ds(i * 16, 16)]
                pv = p_v[pl.ds(i * 16, 16)]
                return jnp.maximum(pm, jnp.where(rv >= rcv, pv, -1.0))

            pmv = lax.fori_loop(0, NSV, ib, jnp.full((16,), -1.0, jnp.float32))
            pm = jnp.max(pmv)
            use = jnp.logical_and(rc < BIG, rc >= 1)
            return ap + jnp.where(use, pm, 0.0)

        ap = lax.fori_loop(0, NSLOT, pb, jnp.float32(0.0))
        out_v[...] = jnp.full((16,), ap * jnp.float32(1.0 / NLBL),
                              jnp.float32)
        pltpu.sync_copy(out_v, ap_hbm)


@functools.lru_cache(maxsize=1)
def _build():
    mesh = plsc.VectorSubcoreMesh(
        core_axis_name="c", subcore_axis_name="s",
        num_cores=NCORES, num_subcores=NSUB)
    k1 = pl.kernel(
        _k1_body, mesh=mesh,
        compiler_params=pltpu.CompilerParams(needs_layout_passes=False),
        out_type=jax.ShapeDtypeStruct((ML * CAP,), jnp.int32),
        scratch_types=[pltpu.VMEM((N,), jnp.float32),
                       pltpu.VMEM((N,), jnp.float32),
                       pltpu.VMEM((LPW * 16,), jnp.float32),
                       pltpu.VMEM((LPW * 16,), jnp.float32),
                       pltpu.VMEM((LPW * CAP,), jnp.int32)])
    k2 = pl.kernel(
        _k2_body, mesh=mesh,
        compiler_params=pltpu.CompilerParams(needs_layout_passes=False),
        out_type=[jax.ShapeDtypeStruct((ML * 16,), jnp.int32),
                  jax.ShapeDtypeStruct((ML * 16,), jnp.float32)],
        scratch_types=[pltpu.VMEM((ML * CAP,), jnp.int32),
                       pltpu.VMEM((N,), jnp.float32),
                       pltpu.VMEM((N + 16,), jnp.int32),
                       pltpu.VMEM((ML * 16,), jnp.int32),
                       pltpu.VMEM((ML * 16,), jnp.float32)])
    k3 = pl.kernel(
        _k3_body, mesh=mesh,
        compiler_params=pltpu.CompilerParams(needs_layout_passes=False),
        out_type=jax.ShapeDtypeStruct((NSLOT,), jnp.int32),
        scratch_types=[pltpu.VMEM((N,), jnp.float32),
                       pltpu.VMEM((LPW * 16,), jnp.int32),
                       pltpu.VMEM((LPW * 16,), jnp.float32),
                       pltpu.VMEM((8,), jnp.int32)])
    k4 = pl.kernel(
        _k4_body, mesh=mesh,
        compiler_params=pltpu.CompilerParams(needs_layout_passes=False),
        out_type=jax.ShapeDtypeStruct((16,), jnp.float32),
        scratch_types=[pltpu.VMEM((NSLOT,), jnp.int32),
                       pltpu.VMEM((NSLOT,), jnp.float32),
                       pltpu.VMEM((16,), jnp.float32)])
    return k1, k2, k3, k4


def kernel(scores, segments, labels):
    smin = segments[:, 0]
    smax = segments[:, 1]
    lmin = jnp.concatenate(
        [labels[:, 0], jnp.full((ML - NLBL,), -2.0, jnp.float32)])
    lmax = jnp.concatenate(
        [labels[:, 1], jnp.full((ML - NLBL,), -1.0, jnp.float32)])
    lmin16 = jnp.broadcast_to(lmin[:, None], (ML, 16)).reshape(ML * 16)
    lmax16 = jnp.broadcast_to(lmax[:, None], (ML, 16)).reshape(ML * 16)

    k1, k2, k3, k4 = _build()
    lists = k1(smin, smax, lmin16, lmax16)
    chosen, svals = k2(lists, scores)
    ranks = k3(scores, chosen, svals)
    ap16 = k4(ranks)
    return ap16[0]


# speedup vs baseline: 10.8409x; 1.0166x over previous
import functools

import jax
import jax.numpy as jnp
from jax import lax
from jax.experimental import pallas as pl
from jax.experimental.pallas import tpu as pltpu
from jax.experimental.pallas import tpu_sc as plsc

N = 20000
NV = N // 16
NLBL = 200
NCORES = 2
NSUB = 16
NW = NCORES * NSUB
LPW = 7
ML = NW * LPW
MLV = ML // 16
CAP = 224
K = 200
BIG = 1 << 30
NSLOT = NW * 8
NSV = NSLOT // 16

def _wid():
    return lax.axis_index("s") * NCORES + lax.axis_index("c")


def _k1_body(smin_hbm, smax_hbm, lmin_hbm, lmax_hbm, lists_hbm,
             smin_v, smax_v, lmin_v, lmax_v, list_v):
    w = _wid()
    base_l = w * LPW
    pltpu.sync_copy(smin_hbm, smin_v)
    pltpu.sync_copy(smax_hbm, smax_v)
    pltpu.sync_copy(lmin_hbm.at[pl.ds(base_l * 16, LPW * 16)], lmin_v)
    pltpu.sync_copy(lmax_hbm.at[pl.ds(base_l * 16, LPW * 16)], lmax_v)
    lanes = lax.iota(jnp.int32, 16)

    bigv = jnp.full((16,), BIG, jnp.int32)

    def initb(i, carry):
        list_v[pl.ds(i * 16, 16)] = bigv
        return carry

    lax.fori_loop(0, LPW * CAP // 16, initb, 0)
    bmin = [lmin_v[pl.ds(j * 16, 16)] for j in range(LPW)]
    bmax = [lmax_v[pl.ds(j * 16, 16)] for j in range(LPW)]
    blen = [bmax[j] - bmin[j] for j in range(LPW)]
    kcap = [jnp.full((16,), j * CAP + K - 1, jnp.int32) for j in range(LPW)]

    def body(i, offs):
        b = i * 16
        sm = smin_v[pl.ds(b, 16)]
        sx = smax_v[pl.ds(b, 16)]
        la = sx - sm
        iv = lanes + b
        new_offs = []
        for j in range(LPW):
            raw = jnp.minimum(sx, bmax[j]) - jnp.maximum(sm, bmin[j])
            m = (raw * 3.0) > (la + blen[j])
            off = offs[j]
            mstore = jnp.logical_and(m, off < kcap[j])
            mi = mstore.astype(jnp.int32)
            dest = jnp.minimum(off, kcap[j]) + plsc.cumsum(mi)
            plsc.store_scatter(list_v, [dest], iv, mask=mstore)
            new_offs.append(off + plsc.all_reduce_population_count(m))
        return tuple(new_offs)

    lax.fori_loop(
        0, NV, body,
        tuple(jnp.full((16,), j * CAP - 1, jnp.int32) for j in range(LPW)))
    pltpu.sync_copy(list_v, lists_hbm.at[pl.ds(base_l * CAP, LPW * CAP)])


def _k2_body(lists_hbm, scores_hbm, chosen_hbm, svals_hbm,
             lists_v, scores_v, taken_v, chosen_v, svals_v):
    w = _wid()

    @pl.when(w == 0)
    def _():
        pltpu.sync_copy(lists_hbm, lists_v)
        pltpu.sync_copy(scores_hbm, scores_v)
        lanes = lax.iota(jnp.int32, 16)
        zeros = jnp.zeros((16,), jnp.int32)
        ones = jnp.ones((16,), jnp.int32)
        nvec = jnp.full((16,), N, jnp.int32)

        def zbody(i, carry):
            taken_v[pl.ds(i * 16, 16)] = zeros
            return carry

        lax.fori_loop(0, NV + 1, zbody, 0)

        lane0 = lanes == 0
        s15 = jnp.full((16,), 15, jnp.int32)

        def lbody(c, carry):
            def scan_vreg(i):
                candv = lists_v[pl.ds(c * CAP + i * 16, 16)]
                csafe = jnp.minimum(candv, nvec)
                tk = plsc.load_gather(taken_v, [csafe])
                avail = jnp.logical_and(tk == 0, candv < nvec)
                f = plsc.all_reduce_ffs(avail)
                chn = candv.at[jnp.minimum(f, s15)].get(
                    mode="promise_in_bounds")
                return jnp.where(f < 16, chn, -1), candv

            chv, candv0 = scan_vreg(jnp.int32(0))
            last_real = candv0.at[s15].get(mode="promise_in_bounds") < nvec
            pred = jnp.any(jnp.logical_and(chv < 0, last_real))

            def tail():
                def body(i, ch2):
                    chn, _ = scan_vreg(i)
                    return jnp.where(ch2 >= 0, ch2, chn)
                return lax.fori_loop(1, (K + 15) // 16, body,
                                     jnp.full((16,), -1, jnp.int32))

            chv = lax.cond(pred, tail, lambda: chv)
            has = chv >= 0
            csafe = jnp.maximum(chv, 0)
            plsc.store_scatter(taken_v, [csafe], ones,
                               mask=jnp.logical_and(lane0, has))
            sv = plsc.load_gather(scores_v, [csafe])
            chosen_v[pl.ds(c * 16, 16)] = chv
            svals_v[pl.ds(c * 16, 16)] = jnp.where(has, sv, 0.0)
            return carry

        lax.fori_loop(0, ML, lbody, 0)
        pltpu.sync_copy(chosen_v, chosen_hbm)
        pltpu.sync_copy(svals_v, svals_hbm)


def _k3_body(scores_hbm, chosen_hbm, svals_hbm, ranks_hbm,
             scores_v, ch_v, sv_v, rk_v):
    w = _wid()
    pltpu.sync_copy(scores_hbm, scores_v)
    pltpu.sync_copy(chosen_hbm.at[pl.ds(w * LPW * 16, LPW * 16)], ch_v)
    pltpu.sync_copy(svals_hbm.at[pl.ds(w * LPW * 16, LPW * 16)], sv_v)
    lanes = lax.iota(jnp.int32, 16)
    jv = [ch_v[pl.ds(j * 16, 16)] for j in range(LPW)]
    sv = [sv_v[pl.ds(j * 16, 16)] for j in range(LPW)]

    def body(i, accs):
        b = i * 16
        sc = scores_v[pl.ds(b, 16)]
        idxv = lanes + b
        return tuple(
            accs[j] + jnp.logical_or(
                sc > sv[j],
                jnp.logical_and(sc == sv[j], idxv < jv[j])
            ).astype(jnp.int32)
            for j in range(LPW))

    accs = lax.fori_loop(
        0, NV, body, tuple(jnp.zeros((16,), jnp.int32) for _ in range(LPW)))
    lane0 = lanes == 0
    for j in range(LPW + 1):
        if j < LPW:
            rank = jnp.where(jnp.max(jv[j]) >= 0, jnp.sum(accs[j]), BIG)
        else:
            rank = jnp.int32(BIG)
        plsc.store_scatter(rk_v, [jnp.full((16,), j, jnp.int32)],
                           jnp.full((16,), rank, jnp.int32), mask=lane0)
    pltpu.sync_copy(rk_v, ranks_hbm.at[pl.ds(w * 8, 8)])


def _k4_body(ranks_hbm, ap_hbm, rk_v, p_v, out_v):
    w = _wid()

    @pl.when(w == 0)
    def _():
        pltpu.sync_copy(ranks_hbm, rk_v)
        lanes = lax.iota(jnp.int32, 16)
        lane0 = lanes == 0

        def pa(c, carry):
            cv = jnp.full((16,), c, jnp.int32)
            rcv = plsc.load_gather(rk_v, [cv])
            rc = jnp.max(rcv)

            def ib(i, acc):
                rv = rk_v[pl.ds(i * 16, 16)]
                return acc + (rv < rcv).astype(jnp.int32)

            acc = lax.fori_loop(0, NSV, ib, jnp.zeros((16,), jnp.int32))
            pos = jnp.sum(acc)
            valid = rc < BIG
            pvec = (jnp.full((16,), pos, jnp.float32) + 1.0) / \
                   (rcv.astype(jnp.float32) + 1.0)
            pvec = jnp.where(valid, pvec, -1.0)
            plsc.store_scatter(p_v, [cv], pvec, mask=lane0)
            return carry

        lax.fori_loop(0, NSLOT, pa, 0)

        def pb(c, ap):
            cv = jnp.full((16,), c, jnp.int32)
            rcv = plsc.load_gather(rk_v, [cv])
            rc = jnp.max(rcv)

            def ib(i, pm):
                rv = rk_v[pl.ds(i * 16, 16)]
                pv = p_v[pl.ds(i * 16, 16)]
                return jnp.maximum(pm, jnp.where(rv >= rcv, pv, -1.0))

            pmv = lax.fori_loop(0, NSV, ib, jnp.full((16,), -1.0, jnp.float32))
            pm = jnp.max(pmv)
            use = jnp.logical_and(rc < BIG, rc >= 1)
            return ap + jnp.where(use, pm, 0.0)

        ap = lax.fori_loop(0, NSLOT, pb, jnp.float32(0.0))
        out_v[...] = jnp.full((16,), ap * jnp.float32(1.0 / NLBL),
                              jnp.float32)
        pltpu.sync_copy(out_v, ap_hbm)


@functools.lru_cache(maxsize=1)
def _build():
    mesh = plsc.VectorSubcoreMesh(
        core_axis_name="c", subcore_axis_name="s",
        num_cores=NCORES, num_subcores=NSUB)
    k1 = pl.kernel(
        _k1_body, mesh=mesh,
        compiler_params=pltpu.CompilerParams(needs_layout_passes=False),
        out_type=jax.ShapeDtypeStruct((ML * CAP,), jnp.int32),
        scratch_types=[pltpu.VMEM((N,), jnp.float32),
                       pltpu.VMEM((N,), jnp.float32),
                       pltpu.VMEM((LPW * 16,), jnp.float32),
                       pltpu.VMEM((LPW * 16,), jnp.float32),
                       pltpu.VMEM((LPW * CAP,), jnp.int32)])
    k2 = pl.kernel(
        _k2_body, mesh=mesh,
        compiler_params=pltpu.CompilerParams(needs_layout_passes=False),
        out_type=[jax.ShapeDtypeStruct((ML * 16,), jnp.int32),
                  jax.ShapeDtypeStruct((ML * 16,), jnp.float32)],
        scratch_types=[pltpu.VMEM((ML * CAP,), jnp.int32),
                       pltpu.VMEM((N,), jnp.float32),
                       pltpu.VMEM((N + 16,), jnp.int32),
                       pltpu.VMEM((ML * 16,), jnp.int32),
                       pltpu.VMEM((ML * 16,), jnp.float32)])
    k3 = pl.kernel(
        _k3_body, mesh=mesh,
        compiler_params=pltpu.CompilerParams(needs_layout_passes=False),
        out_type=jax.ShapeDtypeStruct((NSLOT,), jnp.int32),
        scratch_types=[pltpu.VMEM((N,), jnp.float32),
                       pltpu.VMEM((LPW * 16,), jnp.int32),
                       pltpu.VMEM((LPW * 16,), jnp.float32),
                       pltpu.VMEM((8,), jnp.int32)])
    k4 = pl.kernel(
        _k4_body, mesh=mesh,
        compiler_params=pltpu.CompilerParams(needs_layout_passes=False),
        out_type=jax.ShapeDtypeStruct((16,), jnp.float32),
        scratch_types=[pltpu.VMEM((NSLOT,), jnp.int32),
                       pltpu.VMEM((NSLOT,), jnp.float32),
                       pltpu.VMEM((16,), jnp.float32)])
    return k1, k2, k3, k4


def kernel(scores, segments, labels):
    smin = segments[:, 0]
    smax = segments[:, 1]
    lmin = jnp.concatenate(
        [labels[:, 0], jnp.full((ML - NLBL,), -2.0, jnp.float32)])
    lmax = jnp.concatenate(
        [labels[:, 1], jnp.full((ML - NLBL,), -1.0, jnp.float32)])
    lmin16 = jnp.broadcast_to(lmin[:, None], (ML, 16)).reshape(ML * 16)
    lmax16 = jnp.broadcast_to(lmax[:, None], (ML, 16)).reshape(ML * 16)

    k1, k2, k3, k4 = _build()
    lists = k1(smin, smax, lmin16, lmax16)
    chosen, svals = k2(lists, scores)
    ranks = k3(scores, chosen, svals)
    ap16 = k4(ranks)
    return ap16[0]
